# Initial kernel scaffold; baseline (speedup 1.0000x reference)
import jax
import jax.numpy as jnp
from jax.experimental import pallas as pl


def kernel(proposals, box_regression, scores):
    raise NotImplementedError("write your pallas kernel here")


# speedup vs baseline: 5.3249x; 5.3249x over previous
import functools

import jax
import jax.numpy as jnp
from jax import lax
from jax.experimental import pallas as pl
from jax.experimental.pallas import tpu as pltpu
from jax.experimental.pallas import tpu_sc as plsc

_N = 20000
_NS = 16
_CH = 1280
_NP = _NS * _CH
_SL = _CH // 16
_K = 100
_NEG = float("-inf")
_BIG = 3.0e7


def _shuf(v, perm):
    return lax.gather(
        v,
        perm[:, None],
        lax.GatherDimensionNumbers(
            offset_dims=(), collapsed_slice_dims=(0,), start_index_map=(0,)
        ),
        slice_sizes=(1,),
        mode=lax.GatherScatterMode.PROMISE_IN_BOUNDS,
    )


def _tree(op, v, li):
    for sh in (8, 4, 2, 1):
        v = op(v, _shuf(v, jnp.bitwise_xor(li, sh)))
    return v


def _nms_body(inp_hbm, out_hbm,
              px1, py1, px2, py2, d0, d1, d2, d3, sc,
              bx1, by1, bx2, by2, ar, sw,
              all16, recv, outb, shared):
    s = lax.axis_index("s")
    c = lax.axis_index("c")
    base = s * _CH
    basef = base.astype(jnp.float32)
    li = lax.iota(jnp.int32, 16)
    lif = li.astype(jnp.float32)
    idx00 = basef + lif

    for f, dst in enumerate((px1, py1, px2, py2, d0, d1, d2, d3, sc)):
        pltpu.sync_copy(inp_hbm.at[pl.ds(f * _NP + base, _CH)], dst)

    def dec(i, _):
        o = pl.ds(i * 16, 16)
        x1 = px1[o]
        y1 = py1[o]
        x2 = px2[o]
        y2 = py2[o]
        w = x2 - x1 + 1.0
        h = y2 - y1 + 1.0
        cx = x1 + 0.5 * w
        cy = y1 + 0.5 * h
        dx = d0[o] / 10.0
        dy = d1[o] / 10.0
        dw = jnp.minimum(d2[o] / 5.0, 4.0)
        dh = jnp.minimum(d3[o] / 5.0, 4.0)
        pcx = dx * w + cx
        pcy = dy * h + cy
        pw = jnp.exp(dw) * w
        ph = jnp.exp(dh) * h
        nx1 = jnp.clip(pcx - 0.5 * pw, 0.0, 1023.0)
        ny1 = jnp.clip(pcy - 0.5 * ph, 0.0, 1023.0)
        nx2 = jnp.clip(pcx + 0.5 * pw - 1.0, 0.0, 1023.0)
        ny2 = jnp.clip(pcy + 0.5 * ph - 1.0, 0.0, 1023.0)
        bx1[o] = nx1
        by1[o] = ny1
        bx2[o] = nx2
        by2[o] = ny2
        ar[o] = (nx2 - nx1 + 1.0) * (ny2 - ny1 + 1.0)
        sv = sc[o]
        sw[o] = jnp.where(sv > 0.05, sv, _NEG)
        return 0

    lax.fori_loop(0, _SL, dec, 0)

    def pass_fn(wx1, wy1, wx2, wy2, war):
        def step(i, carry):
            m, vx = carry
            o = pl.ds(i * 16, 16)
            swv = sw[o]
            ix1 = jnp.maximum(wx1, bx1[o])
            iy1 = jnp.maximum(wy1, by1[o])
            ix2 = jnp.minimum(wx2, bx2[o])
            iy2 = jnp.minimum(wy2, by2[o])
            iw = jnp.maximum(ix2 - ix1 + 1.0, 0.0)
            ih = jnp.maximum(iy2 - iy1 + 1.0, 0.0)
            inter = iw * ih
            iou = inter / (war + ar[o] - inter)
            swv = jnp.where(iou > 0.5, _NEG, swv)
            sw[o] = swv
            idxg = idx00 + (i * 16).astype(jnp.float32)
            upd = swv > m
            m = jnp.where(upd, swv, m)
            vx = jnp.where(upd, idxg, vx)
            return (m, vx)

        m0 = jnp.full((16,), _NEG, jnp.float32)
        return lax.fori_loop(0, _SL, step, (m0, idx00))

    fake = jnp.full((16,), _BIG, jnp.float32)
    one = jnp.full((16,), 1.0, jnp.float32)
    m, vx = pass_fn(fake, fake, -fake, -fake, one)

    def outer(k, carry):
        m, vx = carry
        lm = _tree(jnp.maximum, m, li)
        lidx = _tree(jnp.minimum, jnp.where(m == lm, vx, _BIG), li)
        offb = (lidx - basef).astype(jnp.int32)
        gx1 = plsc.load_gather(bx1, [offb])
        gy1 = plsc.load_gather(by1, [offb])
        gx2 = plsc.load_gather(bx2, [offb])
        gy2 = plsc.load_gather(by2, [offb])
        gsc = plsc.load_gather(sc, [offb])
        gar = plsc.load_gather(ar, [offb])
        rec = jnp.where(li == 0, lm,
              jnp.where(li == 1, lidx,
              jnp.where(li == 2, gx1,
              jnp.where(li == 3, gy1,
              jnp.where(li == 4, gx2,
              jnp.where(li == 5, gy2,
              jnp.where(li == 6, gsc, gar)))))))
        recv[...] = rec
        buf = (k % 2) * 256
        pltpu.sync_copy(recv, shared.at[pl.ds(buf + s * 16, 16)])
        plsc.subcore_barrier()
        pltpu.sync_copy(shared.at[pl.ds(buf, 256)], all16)
        maxv = plsc.load_gather(all16, [li * 16])
        idxv = plsc.load_gather(all16, [li * 16 + 1])
        mg = _tree(jnp.maximum, maxv, li)
        cand = jnp.where(maxv == mg, idxv, _BIG)
        bidx = _tree(jnp.minimum, cand, li)
        tstar = _tree(
            jnp.minimum, jnp.where(cand == bidx, lif, _BIG), li
        ).astype(jnp.int32)
        gb = tstar * 16
        wx1 = plsc.load_gather(all16, [gb + 2])
        wy1 = plsc.load_gather(all16, [gb + 3])
        wx2 = plsc.load_gather(all16, [gb + 4])
        wy2 = plsc.load_gather(all16, [gb + 5])
        wsc = plsc.load_gather(all16, [gb + 6])
        war = plsc.load_gather(all16, [gb + 7])
        orec = jnp.where(li == 0, wx1,
               jnp.where(li == 1, wy1,
               jnp.where(li == 2, wx2,
               jnp.where(li == 3, wy2,
               jnp.where(li == 4, wsc, 0.0)))))
        plsc.store_scatter(outb, [k * 16 + li], orec)
        return pass_fn(wx1, wy1, wx2, wy2, war)

    lax.fori_loop(0, _K, outer, (m, vx))

    @pl.when(jnp.logical_and(s == 0, c == 0))
    def _():
        pltpu.sync_copy(outb, out_hbm)


@functools.cache
def _get_nms():
    return pl.kernel(
        _nms_body,
        out_type=jax.ShapeDtypeStruct((_K * 16,), jnp.float32),
        mesh=plsc.VectorSubcoreMesh(
            core_axis_name="c", subcore_axis_name="s", num_cores=2, num_subcores=16
        ),
        scratch_types=(
            [pltpu.VMEM((_CH,), jnp.float32)] * 15
            + [
                pltpu.VMEM((256,), jnp.float32),
                pltpu.VMEM((16,), jnp.float32),
                pltpu.VMEM((_K * 16,), jnp.float32),
                pltpu.VMEM_SHARED((512,), jnp.float32),
            ]
        ),
        compiler_params=pltpu.CompilerParams(needs_layout_passes=False),
    )


@jax.jit
def kernel(proposals, box_regression, scores):
    pt = jnp.pad(proposals, ((0, _NP - _N), (0, 0))).T
    bt = jnp.pad(box_regression, ((0, _NP - _N), (0, 0))).T
    st = jnp.pad(scores, (0, _NP - _N))[None]
    inp = jnp.concatenate([pt, bt, st], axis=0).reshape(-1)
    out = _get_nms()(inp)
    return out.reshape(_K, 16)[:, :5]


# speedup vs baseline: 12.2143x; 2.2938x over previous
import functools

import jax
import jax.numpy as jnp
from jax import lax
from jax.experimental import pallas as pl
from jax.experimental.pallas import tpu as pltpu
from jax.experimental.pallas import tpu_sc as plsc

_N = 20000
_NS = 16
_CH = 1280
_NP = _NS * _CH
_SL = _CH // 16
_K = 100
_NEG = float("-inf")
_BIG = 3.0e7


def _shuf(v, perm):
    return lax.gather(
        v,
        perm[:, None],
        lax.GatherDimensionNumbers(
            offset_dims=(), collapsed_slice_dims=(0,), start_index_map=(0,)
        ),
        slice_sizes=(1,),
        mode=lax.GatherScatterMode.PROMISE_IN_BOUNDS,
    )


def _tree(op, v, li):
    for sh in (8, 4, 2, 1):
        v = op(v, _shuf(v, jnp.bitwise_xor(li, sh)))
    return v


def _nms_body(inp_hbm, out_hbm,
              px1, py1, px2, py2, d0, d1, d2, d3, sc,
              bx1, by1, bx2, by2, ar, sw,
              all16, recv, outb, shared):
    s = lax.axis_index("s")
    c = lax.axis_index("c")
    base = s * _CH
    basef = base.astype(jnp.float32)
    li = lax.iota(jnp.int32, 16)
    lif = li.astype(jnp.float32)
    idx00 = basef + lif

    for f, dst in enumerate((px1, py1, px2, py2, d0, d1, d2, d3, sc)):
        pltpu.sync_copy(inp_hbm.at[pl.ds(f * _NP + base, _CH)], dst)

    def dec(i, _):
        o = pl.ds(i * 16, 16)
        x1 = px1[o]
        y1 = py1[o]
        x2 = px2[o]
        y2 = py2[o]
        w = x2 - x1 + 1.0
        h = y2 - y1 + 1.0
        cx = x1 + 0.5 * w
        cy = y1 + 0.5 * h
        dx = d0[o] / 10.0
        dy = d1[o] / 10.0
        dw = jnp.minimum(d2[o] / 5.0, 4.0)
        dh = jnp.minimum(d3[o] / 5.0, 4.0)
        pcx = dx * w + cx
        pcy = dy * h + cy
        pw = jnp.exp(dw) * w
        ph = jnp.exp(dh) * h
        nx1 = jnp.clip(pcx - 0.5 * pw, 0.0, 1023.0)
        ny1 = jnp.clip(pcy - 0.5 * ph, 0.0, 1023.0)
        nx2 = jnp.clip(pcx + 0.5 * pw - 1.0, 0.0, 1023.0)
        ny2 = jnp.clip(pcy + 0.5 * ph - 1.0, 0.0, 1023.0)
        bx1[o] = nx1
        by1[o] = ny1
        bx2[o] = nx2
        by2[o] = ny2
        ar[o] = (nx2 - nx1 + 1.0) * (ny2 - ny1 + 1.0)
        sv = sc[o]
        sw[o] = jnp.where(sv > 0.05, sv, _NEG)
        return 0

    lax.fori_loop(0, _SL, dec, 0)

    _UNROLL = 4

    def pass_fn(wx1, wy1, wx2, wy2, war):
        def step(i, carry):
            m, vx = carry
            for u in range(_UNROLL):
                o = pl.ds(i * (16 * _UNROLL) + u * 16, 16)
                swv = sw[o]
                ix1 = jnp.maximum(wx1, bx1[o])
                iy1 = jnp.maximum(wy1, by1[o])
                ix2 = jnp.minimum(wx2, bx2[o])
                iy2 = jnp.minimum(wy2, by2[o])
                iw = jnp.maximum(ix2 - ix1 + 1.0, 0.0)
                ih = jnp.maximum(iy2 - iy1 + 1.0, 0.0)
                inter = iw * ih
                iou = inter / (war + ar[o] - inter)
                swv = jnp.where(iou > 0.5, _NEG, swv)
                sw[o] = swv
                idxg = idx00 + (i * (16 * _UNROLL) + u * 16).astype(jnp.float32)
                upd = swv > m
                m = jnp.where(upd, swv, m)
                vx = jnp.where(upd, idxg, vx)
            return (m, vx)

        m0 = jnp.full((16,), _NEG, jnp.float32)
        return lax.fori_loop(0, _SL // _UNROLL, step, (m0, idx00))

    fake = jnp.full((16,), _BIG, jnp.float32)
    one = jnp.full((16,), 1.0, jnp.float32)
    m, vx = pass_fn(fake, fake, -fake, -fake, one)

    def outer(k, carry):
        m, vx = carry
        lm = _tree(jnp.maximum, m, li)
        lidx = _tree(jnp.minimum, jnp.where(m == lm, vx, _BIG), li)
        offb = (lidx - basef).astype(jnp.int32)
        gx1 = plsc.load_gather(bx1, [offb])
        gy1 = plsc.load_gather(by1, [offb])
        gx2 = plsc.load_gather(bx2, [offb])
        gy2 = plsc.load_gather(by2, [offb])
        gsc = plsc.load_gather(sc, [offb])
        gar = plsc.load_gather(ar, [offb])
        rec = jnp.where(li == 0, lm,
              jnp.where(li == 1, lidx,
              jnp.where(li == 2, gx1,
              jnp.where(li == 3, gy1,
              jnp.where(li == 4, gx2,
              jnp.where(li == 5, gy2,
              jnp.where(li == 6, gsc, gar)))))))
        recv[...] = rec
        buf = (k % 2) * 256
        pltpu.sync_copy(recv, shared.at[pl.ds(buf + s * 16, 16)])
        plsc.subcore_barrier()
        pltpu.sync_copy(shared.at[pl.ds(buf, 256)], all16)
        maxv = plsc.load_gather(all16, [li * 16])
        idxv = plsc.load_gather(all16, [li * 16 + 1])
        mg = _tree(jnp.maximum, maxv, li)
        cand = jnp.where(maxv == mg, idxv, _BIG)
        bidx = _tree(jnp.minimum, cand, li)
        tstar = _tree(
            jnp.minimum, jnp.where(cand == bidx, lif, _BIG), li
        ).astype(jnp.int32)
        gb = tstar * 16
        wx1 = plsc.load_gather(all16, [gb + 2])
        wy1 = plsc.load_gather(all16, [gb + 3])
        wx2 = plsc.load_gather(all16, [gb + 4])
        wy2 = plsc.load_gather(all16, [gb + 5])
        wsc = plsc.load_gather(all16, [gb + 6])
        war = plsc.load_gather(all16, [gb + 7])
        orec = jnp.where(li == 0, wx1,
               jnp.where(li == 1, wy1,
               jnp.where(li == 2, wx2,
               jnp.where(li == 3, wy2,
               jnp.where(li == 4, wsc, 0.0)))))
        plsc.store_scatter(outb, [k * 16 + li], orec)
        return pass_fn(wx1, wy1, wx2, wy2, war)

    lax.fori_loop(0, _K, outer, (m, vx))

    @pl.when(jnp.logical_and(s == 0, c == 0))
    def _():
        pltpu.sync_copy(outb, out_hbm)


@functools.cache
def _get_nms():
    return pl.kernel(
        _nms_body,
        out_type=jax.ShapeDtypeStruct((_K * 16,), jnp.float32),
        mesh=plsc.VectorSubcoreMesh(
            core_axis_name="c", subcore_axis_name="s", num_cores=2, num_subcores=16
        ),
        scratch_types=(
            [pltpu.VMEM((_CH,), jnp.float32)] * 15
            + [
                pltpu.VMEM((256,), jnp.float32),
                pltpu.VMEM((16,), jnp.float32),
                pltpu.VMEM((_K * 16,), jnp.float32),
                pltpu.VMEM_SHARED((512,), jnp.float32),
            ]
        ),
        compiler_params=pltpu.CompilerParams(needs_layout_passes=False),
    )


@jax.jit
def kernel(proposals, box_regression, scores):
    pt = jnp.pad(proposals, ((0, _NP - _N), (0, 0))).T
    bt = jnp.pad(box_regression, ((0, _NP - _N), (0, 0))).T
    st = jnp.pad(scores, (0, _NP - _N))[None]
    inp = jnp.concatenate([pt, bt, st], axis=0).reshape(-1)
    out = _get_nms()(inp)
    return out.reshape(_K, 16)[:, :5]
